# baseline (speedup 1.0000x reference)
import functools

import jax
import jax.numpy as jnp
from jax import lax
from jax.experimental import pallas as pl
from jax.experimental.pallas import tpu as pltpu
from jax.experimental.pallas import tpu_sc as plsc

_B = 16384
_D = 64
_CH = 128

_NC, _NS = 2, 16
_NW = _NC * _NS
_BPW = _B // _NW
_NCH = _BPW // _CH

@functools.cache
def _build_sc_gather():
    mesh = plsc.VectorSubcoreMesh(
        core_axis_name="c", subcore_axis_name="s",
        num_cores=_NC, num_subcores=_NS)

    @functools.partial(
        pl.kernel,
        mesh=mesh,
        compiler_params=pltpu.CompilerParams(use_tc_tiling_on_sc=False),
        out_type=[
            jax.ShapeDtypeStruct((_B, _D), jnp.float32),
            jax.ShapeDtypeStruct((_B, _D), jnp.float32),
        ],
        scratch_types=[
            pltpu.VMEM((_NCH, _CH), jnp.int32),
            pltpu.VMEM((_NCH, _CH), jnp.int32),
            pltpu.VMEM((_BPW, _D), jnp.float32),
            pltpu.VMEM((_BPW, _D), jnp.float32),
            pltpu.SemaphoreType.DMA,
        ],
    )
    def sc_gather(uid_hbm, rid_hbm, utab_hbm, rtab_hbm, uout_hbm, rout_hbm,
                  uidx, ridx, urows, rrows, sem):
        wid = lax.axis_index("s") * _NC + lax.axis_index("c")
        base = wid * _BPW
        pltpu.sync_copy(uid_hbm.at[pl.ds(wid * _NCH, _NCH)], uidx)
        pltpu.sync_copy(rid_hbm.at[pl.ds(wid * _NCH, _NCH)], ridx)
        copies = []
        for c in range(_NCH):
            copies.append(pltpu.async_copy(
                utab_hbm.at[uidx.at[c]], urows.at[pl.ds(c * _CH, _CH)], sem))
            copies.append(pltpu.async_copy(
                rtab_hbm.at[ridx.at[c]], rrows.at[pl.ds(c * _CH, _CH)], sem))
        for cp in copies:
            cp.wait()
        pltpu.sync_copy(urows, uout_hbm.at[pl.ds(base, _BPW)])
        pltpu.sync_copy(rrows, rout_hbm.at[pl.ds(base, _BPW)])

    return sc_gather


_BLK = 2048


def _mlp_body(u_ref, r_ref, w1u_ref, w1r_ref, b1_ref, w2_ref, b2_ref,
              w3_ref, b3_ref, o_ref):
    h = jnp.dot(u_ref[...], w1u_ref[...], preferred_element_type=jnp.float32)
    h = h + jnp.dot(r_ref[...], w1r_ref[...],
                    preferred_element_type=jnp.float32)
    h = jnp.maximum(h + b1_ref[...], 0.0)
    h = jnp.maximum(
        jnp.dot(h, w2_ref[...], preferred_element_type=jnp.float32)
        + b2_ref[...], 0.0)
    o_ref[...] = (jnp.dot(h, w3_ref[...], preferred_element_type=jnp.float32)
                  + b3_ref[...])


def _mlp(u_emb, r_emb, w1u, w1r, b1, w2, b2, w3, b3):
    return pl.pallas_call(
        _mlp_body,
        grid=(_B // _BLK,),
        in_specs=[
            pl.BlockSpec((_BLK, _D), lambda i: (i, 0)),
            pl.BlockSpec((_BLK, _D), lambda i: (i, 0)),
            pl.BlockSpec((_D, 64), lambda i: (0, 0)),
            pl.BlockSpec((_D, 64), lambda i: (0, 0)),
            pl.BlockSpec((1, 64), lambda i: (0, 0)),
            pl.BlockSpec((64, 32), lambda i: (0, 0)),
            pl.BlockSpec((1, 32), lambda i: (0, 0)),
            pl.BlockSpec((32, 1), lambda i: (0, 0)),
            pl.BlockSpec((1, 1), lambda i: (0, 0)),
        ],
        out_specs=pl.BlockSpec((_BLK, 1), lambda i: (i, 0)),
        out_shape=jax.ShapeDtypeStruct((_B, 1), jnp.float32),
    )(u_emb, r_emb, w1u, w1r, b1, w2, b2, w3, b3)


def kernel(user_id, restaurant_id, user_table, restaurant_table,
           W1, b1, W2, b2, W3, b3):
    uid2 = user_id.reshape(_B // _CH, _CH)
    rid2 = restaurant_id.reshape(_B // _CH, _CH)
    u_emb, r_emb = _build_sc_gather()(uid2, rid2, user_table, restaurant_table)
    return _mlp(
        u_emb, r_emb,
        W1[:, :_D].T, W1[:, _D:].T, b1.reshape(1, 64),
        W2.T, b2.reshape(1, 32),
        W3.T, b3.reshape(1, 1))


# speedup vs baseline: 3.7986x; 3.7986x over previous
import functools

import jax
import jax.numpy as jnp
from jax import lax
from jax.experimental import pallas as pl
from jax.experimental.pallas import tpu as pltpu
from jax.experimental.pallas import tpu_sc as plsc

_B = 16384
_D = 64
_L = 16

_NC, _NS = 2, 16
_NW = _NC * _NS
_HPW = _B // _NW
_RING = 4
_FL = 64
_NFL = _HPW // _FL


def _sread(ref, idx):
    return ref[pl.ds(idx, _L)][0]


def _gather_pass(tab, si, tcol, sj2, ustart, colbuf, stage, out_hbm,
                 sem_col, sem_out, half):
    lane = lax.broadcasted_iota(jnp.int32, (_L,), 0)

    zero = jnp.zeros((_L,), jnp.float32)

    def zero_body(n, c):
        stage[lax.bitwise_and(n, 1), lax.bitwise_and(n >> 1, _FL - 1),
              pl.ds((1 - half) * _D + (n >> 7) * _L, _L)] = zero
        return c

    lax.fori_loop(0, 2 * _FL * (_D // _L), zero_body, 0)

    tcol[pl.ds(0, _L)] = jnp.full((_L,), -1, jnp.int32)
    for v in range(_HPW // _L):
        tcol[pl.ds(_L + v * _L, _L)] = si[pl.ds(v * _L, _L)] >> 7
    nu = 0
    for v in range(_HPW // _L):
        cur = tcol[pl.ds(_L + v * _L, _L)]
        prev = tcol[pl.ds(_L + v * _L - 1, _L)]
        m = cur != prev
        plsc.store_compressed(ustart.at[pl.ds(nu, _L)], v * _L + lane,
                              mask=m)
        nu = nu + plsc.all_reduce_population_count(m)[0]
    ustart[pl.ds(nu, _L)] = jnp.full((_L,), _HPW, jnp.int32)

    def fire_col(u, slot):
        t = _sread(si, _sread(ustart, u)) >> 7
        start = pl.multiple_of(t * 128, 128)
        pltpu.async_copy(tab.at[:, pl.ds(start, 128)],
                         colbuf.at[slot], sem_col)

    def prime(u, c):
        @pl.when(u < nu)
        def _():
            fire_col(u, u)
        return c

    lax.fori_loop(0, _RING, prime, 0)

    def col_body(u, cc):
        slot = lax.rem(u, _RING)
        pltpu.make_async_copy(tab.at[:, pl.ds(0, 128)],
                              colbuf.at[slot], sem_col).wait()
        slot_v = jnp.full((_L,), slot, jnp.int32)

        def hit_body(h, hc):
            i = _sread(si, h)
            l_v = jnp.full((_L,), lax.bitwise_and(i, 127), jnp.int32)
            fb = h // _FL
            buf = lax.rem(fb, 2)
            r = lax.rem(h, _FL)
            buf_v = jnp.full((_L,), buf, jnp.int32)
            r_v = jnp.full((_L,), r, jnp.int32)
            for k in range(_D // _L):
                c = k * _L + lane
                vals = plsc.load_gather(colbuf, [slot_v, c, l_v])
                plsc.store_scatter(stage, [buf_v, r_v, half * _D + c], vals)

            @pl.when(r == _FL - 1)
            def _():
                @pl.when(fb >= 2)
                def _():
                    pltpu.make_async_copy(
                        stage.at[buf], out_hbm.at[sj2.at[0]], sem_out).wait()

                pltpu.async_copy(stage.at[buf], out_hbm.at[sj2.at[fb]],
                                 sem_out)

            return hc

        lax.fori_loop(_sread(ustart, u), _sread(ustart, u + 1), hit_body, 0)

        @pl.when(u + _RING < nu)
        def _():
            fire_col(u + _RING, slot)

        return cc

    lax.fori_loop(0, nu, col_body, 0)

    pltpu.make_async_copy(stage.at[0], out_hbm.at[sj2.at[0]], sem_out).wait()
    pltpu.make_async_copy(stage.at[1], out_hbm.at[sj2.at[0]], sem_out).wait()


@functools.cache
def _build_sc_gather():
    mesh = plsc.VectorSubcoreMesh(
        core_axis_name="c", subcore_axis_name="s",
        num_cores=_NC, num_subcores=_NS)

    @functools.partial(
        pl.kernel,
        mesh=mesh,
        compiler_params=pltpu.CompilerParams(needs_layout_passes=False),
        out_type=[
            jax.ShapeDtypeStruct((_B, 2 * _D), jnp.float32),
            jax.ShapeDtypeStruct((_B, 2 * _D), jnp.float32),
        ],
        scratch_types=[
            pltpu.VMEM((_HPW + _L,), jnp.int32),
            pltpu.VMEM((_HPW + 3 * _L,), jnp.int32),
            pltpu.VMEM((_HPW + 3 * _L,), jnp.int32),
            pltpu.VMEM((_NFL, _FL), jnp.int32),
            pltpu.VMEM((_RING, _D, 128), jnp.float32),
            pltpu.VMEM((2, _FL, 2 * _D), jnp.float32),
            pltpu.SemaphoreType.DMA,
            pltpu.SemaphoreType.DMA,
        ],
    )
    def sc_gather(usi_hbm, usj_hbm, rsi_hbm, rsj_hbm, utab_hbm, rtab_hbm,
                  uout_hbm, rout_hbm,
                  si, tcol, ustart, sj2, colbuf, stage, sem_col, sem_out):
        wid = lax.axis_index("s") * _NC + lax.axis_index("c")
        base = wid * _HPW
        pltpu.sync_copy(usi_hbm.at[pl.ds(base, _HPW)], si.at[pl.ds(0, _HPW)])
        pltpu.sync_copy(usj_hbm.at[pl.ds(wid * _NFL, _NFL)], sj2)
        _gather_pass(utab_hbm, si, tcol, sj2, ustart, colbuf, stage,
                     uout_hbm, sem_col, sem_out, 0)
        pltpu.sync_copy(rsi_hbm.at[pl.ds(base, _HPW)], si.at[pl.ds(0, _HPW)])
        pltpu.sync_copy(rsj_hbm.at[pl.ds(wid * _NFL, _NFL)], sj2)
        _gather_pass(rtab_hbm, si, tcol, sj2, ustart, colbuf, stage,
                     rout_hbm, sem_col, sem_out, 1)

    return sc_gather


_BLK = 2048


def _mlp_body(xu_ref, xr_ref, w1_ref, b1_ref, w2_ref, b2_ref, w3_ref,
              b3_ref, o_ref):
    x = xu_ref[...] + xr_ref[...]
    h = jnp.dot(x, w1_ref[...], preferred_element_type=jnp.float32)
    h = jnp.maximum(h + b1_ref[...], 0.0)
    h = jnp.maximum(
        jnp.dot(h, w2_ref[...], preferred_element_type=jnp.float32)
        + b2_ref[...], 0.0)
    o_ref[...] = (jnp.dot(h, w3_ref[...], preferred_element_type=jnp.float32)
                  + b3_ref[...])


def _mlp(xu, xr, w1, b1, w2, b2, w3, b3):
    return pl.pallas_call(
        _mlp_body,
        grid=(_B // _BLK,),
        in_specs=[
            pl.BlockSpec((_BLK, 2 * _D), lambda i: (i, 0)),
            pl.BlockSpec((_BLK, 2 * _D), lambda i: (i, 0)),
            pl.BlockSpec((2 * _D, 64), lambda i: (0, 0)),
            pl.BlockSpec((1, 64), lambda i: (0, 0)),
            pl.BlockSpec((64, 32), lambda i: (0, 0)),
            pl.BlockSpec((1, 32), lambda i: (0, 0)),
            pl.BlockSpec((32, 1), lambda i: (0, 0)),
            pl.BlockSpec((1, 1), lambda i: (0, 0)),
        ],
        out_specs=pl.BlockSpec((_BLK, 1), lambda i: (i, 0)),
        out_shape=jax.ShapeDtypeStruct((_B, 1), jnp.float32),
    )(xu, xr, w1, b1, w2, b2, w3, b3)


def kernel(user_id, restaurant_id, user_table, restaurant_table,
           W1, b1, W2, b2, W3, b3):
    pos = lax.iota(jnp.int32, _B)
    usi, usj = lax.sort_key_val(user_id, pos)
    rsi, rsj = lax.sort_key_val(restaurant_id, pos)
    xu, xr = _build_sc_gather()(
        usi, usj.reshape(_B // _FL, _FL), rsi, rsj.reshape(_B // _FL, _FL),
        user_table.T, restaurant_table.T)
    return _mlp(xu, xr, W1.T, b1.reshape(1, 64), W2.T, b2.reshape(1, 32),
                W3.T, b3.reshape(1, 1))


# speedup vs baseline: 3.8056x; 1.0018x over previous
import functools

import jax
import jax.numpy as jnp
from jax import lax
from jax.experimental import pallas as pl
from jax.experimental.pallas import tpu as pltpu
from jax.experimental.pallas import tpu_sc as plsc

_B = 16384
_D = 64
_L = 16

_NC, _NS = 2, 16
_NW = _NC * _NS
_HPW = _B // _NW
_RING = 4
_FL = 64
_NFL = _HPW // _FL


def _sread(ref, idx):
    return ref[pl.ds(idx, _L)][0]


def _gather_pass(tab, si, tcol, sj2, ustart, colbuf, stage, out_hbm,
                 sem_col, sem_out, half):
    lane = lax.broadcasted_iota(jnp.int32, (_L,), 0)

    tcol[pl.ds(0, _L)] = jnp.full((_L,), -1, jnp.int32)
    for v in range(_HPW // _L):
        tcol[pl.ds(_L + v * _L, _L)] = si[pl.ds(v * _L, _L)] >> 7
    nu = 0
    for v in range(_HPW // _L):
        cur = tcol[pl.ds(_L + v * _L, _L)]
        prev = tcol[pl.ds(_L + v * _L - 1, _L)]
        m = cur != prev
        plsc.store_compressed(ustart.at[pl.ds(nu, _L)], v * _L + lane,
                              mask=m)
        nu = nu + plsc.all_reduce_population_count(m)[0]
    ustart[pl.ds(nu, _L)] = jnp.full((_L,), _HPW, jnp.int32)

    def fire_col(u, slot):
        t = _sread(si, _sread(ustart, u)) >> 7
        start = pl.multiple_of(t * 128, 128)
        pltpu.async_copy(tab.at[:, pl.ds(start, 128)],
                         colbuf.at[slot], sem_col)

    def prime(u, c):
        @pl.when(u < nu)
        def _():
            fire_col(u, u)
        return c

    lax.fori_loop(0, _RING, prime, 0)

    def col_body(u, cc):
        slot = lax.rem(u, _RING)
        pltpu.make_async_copy(tab.at[:, pl.ds(0, 128)],
                              colbuf.at[slot], sem_col).wait()
        slot_v = jnp.full((_L,), slot, jnp.int32)

        def hit_body(h, hc):
            i = _sread(si, h)
            l_v = jnp.full((_L,), lax.bitwise_and(i, 127), jnp.int32)
            r_v = jnp.full((_L,), h, jnp.int32)
            for k in range(_D // _L):
                c = k * _L + lane
                vals = plsc.load_gather(colbuf, [slot_v, c, l_v])
                plsc.store_scatter(stage, [r_v, half * _D + c], vals)

            return hc

        lax.fori_loop(_sread(ustart, u), _sread(ustart, u + 1), hit_body, 0)

        @pl.when(u + _RING < nu)
        def _():
            fire_col(u + _RING, slot)

        return cc

    lax.fori_loop(0, nu, col_body, 0)

    for fb in range(_NFL):
        pltpu.async_copy(stage.at[pl.ds(fb * _FL, _FL)],
                         out_hbm.at[sj2.at[fb]], sem_out)
    for fb in range(_NFL):
        pltpu.make_async_copy(stage.at[pl.ds(fb * _FL, _FL)],
                              out_hbm.at[sj2.at[0]], sem_out).wait()


@functools.cache
def _build_sc_gather():
    mesh = plsc.VectorSubcoreMesh(
        core_axis_name="c", subcore_axis_name="s",
        num_cores=_NC, num_subcores=_NS)

    @functools.partial(
        pl.kernel,
        mesh=mesh,
        compiler_params=pltpu.CompilerParams(needs_layout_passes=False),
        out_type=[
            jax.ShapeDtypeStruct((_B, 2 * _D), jnp.float32),
            jax.ShapeDtypeStruct((_B, 2 * _D), jnp.float32),
        ],
        scratch_types=[
            pltpu.VMEM((_HPW + _L,), jnp.int32),
            pltpu.VMEM((_HPW + 3 * _L,), jnp.int32),
            pltpu.VMEM((_HPW + 3 * _L,), jnp.int32),
            pltpu.VMEM((_NFL, _FL), jnp.int32),
            pltpu.VMEM((_RING, _D, 128), jnp.float32),
            pltpu.VMEM((_HPW, 2 * _D), jnp.float32),
            pltpu.SemaphoreType.DMA,
            pltpu.SemaphoreType.DMA,
        ],
    )
    def sc_gather(usi_hbm, usj_hbm, rsi_hbm, rsj_hbm, utab_hbm, rtab_hbm,
                  uout_hbm, rout_hbm,
                  si, tcol, ustart, sj2, colbuf, stage, sem_col, sem_out):
        wid = lax.axis_index("s") * _NC + lax.axis_index("c")
        base = wid * _HPW
        pltpu.sync_copy(usi_hbm.at[pl.ds(base, _HPW)], si.at[pl.ds(0, _HPW)])
        pltpu.sync_copy(usj_hbm.at[pl.ds(wid * _NFL, _NFL)], sj2)
        _gather_pass(utab_hbm, si, tcol, sj2, ustart, colbuf, stage,
                     uout_hbm, sem_col, sem_out, 0)
        pltpu.sync_copy(rsi_hbm.at[pl.ds(base, _HPW)], si.at[pl.ds(0, _HPW)])
        pltpu.sync_copy(rsj_hbm.at[pl.ds(wid * _NFL, _NFL)], sj2)
        _gather_pass(rtab_hbm, si, tcol, sj2, ustart, colbuf, stage,
                     rout_hbm, sem_col, sem_out, 1)

    return sc_gather


_BLK = 2048


def _mlp_body(xu_ref, xr_ref, w1u_ref, w1r_ref, b1_ref, w2_ref, b2_ref,
              w3_ref, b3_ref, o_ref):
    h = jnp.dot(xu_ref[:, :_D], w1u_ref[...],
                preferred_element_type=jnp.float32)
    h = h + jnp.dot(xr_ref[:, _D:], w1r_ref[...],
                    preferred_element_type=jnp.float32)
    h = jnp.maximum(h + b1_ref[...], 0.0)
    h = jnp.maximum(
        jnp.dot(h, w2_ref[...], preferred_element_type=jnp.float32)
        + b2_ref[...], 0.0)
    o_ref[...] = (jnp.dot(h, w3_ref[...], preferred_element_type=jnp.float32)
                  + b3_ref[...])


def _mlp(xu, xr, w1u, w1r, b1, w2, b2, w3, b3):
    return pl.pallas_call(
        _mlp_body,
        grid=(_B // _BLK,),
        in_specs=[
            pl.BlockSpec((_BLK, 2 * _D), lambda i: (i, 0)),
            pl.BlockSpec((_BLK, 2 * _D), lambda i: (i, 0)),
            pl.BlockSpec((_D, 64), lambda i: (0, 0)),
            pl.BlockSpec((_D, 64), lambda i: (0, 0)),
            pl.BlockSpec((1, 64), lambda i: (0, 0)),
            pl.BlockSpec((64, 32), lambda i: (0, 0)),
            pl.BlockSpec((1, 32), lambda i: (0, 0)),
            pl.BlockSpec((32, 1), lambda i: (0, 0)),
            pl.BlockSpec((1, 1), lambda i: (0, 0)),
        ],
        out_specs=pl.BlockSpec((_BLK, 1), lambda i: (i, 0)),
        out_shape=jax.ShapeDtypeStruct((_B, 1), jnp.float32),
    )(xu, xr, w1u, w1r, b1, w2, b2, w3, b3)


def kernel(user_id, restaurant_id, user_table, restaurant_table,
           W1, b1, W2, b2, W3, b3):
    pos = lax.iota(jnp.int32, _B)
    usi, usj = lax.sort_key_val(user_id, pos)
    rsi, rsj = lax.sort_key_val(restaurant_id, pos)
    xu, xr = _build_sc_gather()(
        usi, usj.reshape(_B // _FL, _FL), rsi, rsj.reshape(_B // _FL, _FL),
        user_table.T, restaurant_table.T)
    return _mlp(xu, xr, W1[:, :_D].T, W1[:, _D:].T, b1.reshape(1, 64),
                W2.T, b2.reshape(1, 32), W3.T, b3.reshape(1, 1))


# speedup vs baseline: 4.0508x; 1.0644x over previous
import functools

import jax
import jax.numpy as jnp
from jax import lax
from jax.experimental import pallas as pl
from jax.experimental.pallas import tpu as pltpu
from jax.experimental.pallas import tpu_sc as plsc

_B = 16384
_D = 64
_L = 16

_NC, _NS = 2, 16
_NW = _NC * _NS
_HPW = _B // _NW
_RING = 6
_FL = 64
_NFL = _HPW // _FL


def _sread(ref, idx):
    return ref[pl.ds(idx, _L)][0]


def _gather_pass(tab, si, tcol, sj2, ustart, colbuf, stage, out_hbm,
                 sem_col, sem_out, half):
    lane = lax.broadcasted_iota(jnp.int32, (_L,), 0)

    tcol[pl.ds(0, _L)] = jnp.full((_L,), -1, jnp.int32)
    for v in range(_HPW // _L):
        tcol[pl.ds(_L + v * _L, _L)] = si[pl.ds(v * _L, _L)] >> 7
    nu = 0
    for v in range(_HPW // _L):
        cur = tcol[pl.ds(_L + v * _L, _L)]
        prev = tcol[pl.ds(_L + v * _L - 1, _L)]
        m = cur != prev
        plsc.store_compressed(ustart.at[pl.ds(nu, _L)], v * _L + lane,
                              mask=m)
        nu = nu + plsc.all_reduce_population_count(m)[0]
    ustart[pl.ds(nu, _L)] = jnp.full((_L,), _HPW, jnp.int32)

    def fire_col(u, slot):
        t = _sread(si, _sread(ustart, u)) >> 7
        start = pl.multiple_of(t * 128, 128)
        pltpu.async_copy(tab.at[:, pl.ds(start, 128)],
                         colbuf.at[slot], sem_col)

    def prime(u, c):
        @pl.when(u < nu)
        def _():
            fire_col(u, u)
        return c

    lax.fori_loop(0, _RING, prime, 0)

    def col_body(u, cc):
        slot = lax.rem(u, _RING)
        pltpu.make_async_copy(tab.at[:, pl.ds(0, 128)],
                              colbuf.at[slot], sem_col).wait()
        slot_v = jnp.full((_L,), slot, jnp.int32)

        def hit_body(h, hc):
            i = _sread(si, h)
            l_v = jnp.full((_L,), lax.bitwise_and(i, 127), jnp.int32)
            r_v = jnp.full((_L,), h, jnp.int32)
            for k in range(_D // _L):
                c = k * _L + lane
                vals = plsc.load_gather(colbuf, [slot_v, c, l_v])
                plsc.store_scatter(stage, [r_v, half * _D + c], vals)

            return hc

        lax.fori_loop(_sread(ustart, u), _sread(ustart, u + 1), hit_body, 0)

        @pl.when(u + _RING < nu)
        def _():
            fire_col(u + _RING, slot)

        return cc

    lax.fori_loop(0, nu, col_body, 0)

    for fb in range(_NFL):
        pltpu.async_copy(stage.at[pl.ds(fb * _FL, _FL)],
                         out_hbm.at[sj2.at[fb]], sem_out)
    for fb in range(_NFL):
        pltpu.make_async_copy(stage.at[pl.ds(fb * _FL, _FL)],
                              out_hbm.at[sj2.at[0]], sem_out).wait()


@functools.cache
def _build_sc_gather():
    mesh = plsc.VectorSubcoreMesh(
        core_axis_name="c", subcore_axis_name="s",
        num_cores=_NC, num_subcores=_NS)

    @functools.partial(
        pl.kernel,
        mesh=mesh,
        compiler_params=pltpu.CompilerParams(needs_layout_passes=False),
        out_type=[
            jax.ShapeDtypeStruct((_B, 2 * _D), jnp.float32),
            jax.ShapeDtypeStruct((_B, 2 * _D), jnp.float32),
        ],
        scratch_types=[
            pltpu.VMEM((_HPW + _L,), jnp.int32),
            pltpu.VMEM((_HPW + 3 * _L,), jnp.int32),
            pltpu.VMEM((_HPW + 3 * _L,), jnp.int32),
            pltpu.VMEM((_NFL, _FL), jnp.int32),
            pltpu.VMEM((_RING, _D, 128), jnp.float32),
            pltpu.VMEM((_HPW, 2 * _D), jnp.float32),
            pltpu.SemaphoreType.DMA,
            pltpu.SemaphoreType.DMA,
        ],
    )
    def sc_gather(usi_hbm, usj_hbm, rsi_hbm, rsj_hbm, utab_hbm, rtab_hbm,
                  uout_hbm, rout_hbm,
                  si, tcol, ustart, sj2, colbuf, stage, sem_col, sem_out):
        wid = lax.axis_index("s") * _NC + lax.axis_index("c")
        base = wid * _HPW
        pltpu.sync_copy(usi_hbm.at[pl.ds(base, _HPW)], si.at[pl.ds(0, _HPW)])
        pltpu.sync_copy(usj_hbm.at[pl.ds(wid * _NFL, _NFL)], sj2)
        _gather_pass(utab_hbm, si, tcol, sj2, ustart, colbuf, stage,
                     uout_hbm, sem_col, sem_out, 0)
        pltpu.sync_copy(rsi_hbm.at[pl.ds(base, _HPW)], si.at[pl.ds(0, _HPW)])
        pltpu.sync_copy(rsj_hbm.at[pl.ds(wid * _NFL, _NFL)], sj2)
        _gather_pass(rtab_hbm, si, tcol, sj2, ustart, colbuf, stage,
                     rout_hbm, sem_col, sem_out, 1)

    return sc_gather


_BLK = 2048


def _mlp_body(xu_ref, xr_ref, w1u_ref, w1r_ref, b1_ref, w2_ref, b2_ref,
              w3_ref, b3_ref, o_ref):
    h = jnp.dot(xu_ref[:, :_D], w1u_ref[...],
                preferred_element_type=jnp.float32)
    h = h + jnp.dot(xr_ref[:, _D:], w1r_ref[...],
                    preferred_element_type=jnp.float32)
    h = jnp.maximum(h + b1_ref[...], 0.0)
    h = jnp.maximum(
        jnp.dot(h, w2_ref[...], preferred_element_type=jnp.float32)
        + b2_ref[...], 0.0)
    o_ref[...] = (jnp.dot(h, w3_ref[...], preferred_element_type=jnp.float32)
                  + b3_ref[...])


def _mlp(xu, xr, w1u, w1r, b1, w2, b2, w3, b3):
    return pl.pallas_call(
        _mlp_body,
        grid=(_B // _BLK,),
        in_specs=[
            pl.BlockSpec((_BLK, 2 * _D), lambda i: (i, 0)),
            pl.BlockSpec((_BLK, 2 * _D), lambda i: (i, 0)),
            pl.BlockSpec((_D, 64), lambda i: (0, 0)),
            pl.BlockSpec((_D, 64), lambda i: (0, 0)),
            pl.BlockSpec((1, 64), lambda i: (0, 0)),
            pl.BlockSpec((64, 32), lambda i: (0, 0)),
            pl.BlockSpec((1, 32), lambda i: (0, 0)),
            pl.BlockSpec((32, 1), lambda i: (0, 0)),
            pl.BlockSpec((1, 1), lambda i: (0, 0)),
        ],
        out_specs=pl.BlockSpec((_BLK, 1), lambda i: (i, 0)),
        out_shape=jax.ShapeDtypeStruct((_B, 1), jnp.float32),
    )(xu, xr, w1u, w1r, b1, w2, b2, w3, b3)


def kernel(user_id, restaurant_id, user_table, restaurant_table,
           W1, b1, W2, b2, W3, b3):
    pos = lax.iota(jnp.int32, _B)
    usi, usj = lax.sort_key_val(user_id, pos)
    rsi, rsj = lax.sort_key_val(restaurant_id, pos)
    xu, xr = _build_sc_gather()(
        usi, usj.reshape(_B // _FL, _FL), rsi, rsj.reshape(_B // _FL, _FL),
        user_table.T, restaurant_table.T)
    return _mlp(xu, xr, W1[:, :_D].T, W1[:, _D:].T, b1.reshape(1, 64),
                W2.T, b2.reshape(1, 32), W3.T, b3.reshape(1, 1))
